# baseline (speedup 1.0000x reference)
import functools

import jax
import jax.numpy as jnp
from jax import lax
from jax.experimental import pallas as pl
from jax.experimental.pallas import tpu as pltpu
from jax.experimental.pallas import tpu_sc as plsc

B = 16384
DIM = 64
NUM_CORES = 2
NUM_SUBCORES = 16
NW = NUM_CORES * NUM_SUBCORES
RPW = B // NW
LANES = 16
GROUPS = RPW // LANES


def _body(u_hbm, i_hbm, j_hbm, uf_hbm, if_hbm, bias_hbm, out_hbm,
          ui, ii, ji, uv, iv, jv, bi, bj, ov, sem):
    wid = lax.axis_index("s") * NUM_CORES + lax.axis_index("c")
    base = wid * RPW

    pltpu.sync_copy(u_hbm.at[pl.ds(base, RPW)], ui)
    pltpu.sync_copy(i_hbm.at[pl.ds(base, RPW)], ii)
    pltpu.sync_copy(j_hbm.at[pl.ds(base, RPW)], ji)

    c1 = pltpu.async_copy(uf_hbm.at[ui], uv, sem)
    c2 = pltpu.async_copy(if_hbm.at[ii], iv, sem)
    c3 = pltpu.async_copy(if_hbm.at[ji], jv, sem)
    c4 = pltpu.async_copy(bias_hbm.at[ii], bi, sem)
    c5 = pltpu.async_copy(bias_hbm.at[ji], bj, sem)
    c1.wait()
    c2.wait()
    c3.wait()
    c4.wait()
    c5.wait()

    lanes = lax.iota(jnp.int32, LANES)

    def group(g, carry):
        rb = g * LANES
        row_idx = lanes + rb
        acc = bi[pl.ds(rb, LANES)] - bj[pl.ds(rb, LANES)]

        def dstep(d, a):
            cidx = jnp.full((LANES,), d, jnp.int32)
            uu = plsc.load_gather(uv, [row_idx, cidx])
            xi = plsc.load_gather(iv, [row_idx, cidx])
            xj = plsc.load_gather(jv, [row_idx, cidx])
            return a + uu * (xi - xj)

        acc = lax.fori_loop(0, DIM, dstep, acc, unroll=8)
        ov[pl.ds(rb, LANES)] = acc
        return carry

    lax.fori_loop(0, GROUPS, group, 0)
    pltpu.sync_copy(ov, out_hbm.at[pl.ds(base, RPW)])


@functools.partial(jax.jit, static_argnames=())
def kernel(u, i, j, user_factors, item_factors, item_biases):
    mesh = plsc.VectorSubcoreMesh(core_axis_name="c", subcore_axis_name="s")
    k = functools.partial(
        pl.kernel,
        mesh=mesh,
        compiler_params=pltpu.CompilerParams(
            needs_layout_passes=False, use_tc_tiling_on_sc=False),
        out_type=jax.ShapeDtypeStruct((B,), jnp.float32),
        scratch_types=[
            pltpu.VMEM((RPW,), jnp.int32),
            pltpu.VMEM((RPW,), jnp.int32),
            pltpu.VMEM((RPW,), jnp.int32),
            pltpu.VMEM((RPW, DIM), jnp.float32),
            pltpu.VMEM((RPW, DIM), jnp.float32),
            pltpu.VMEM((RPW, DIM), jnp.float32),
            pltpu.VMEM((RPW,), jnp.float32),
            pltpu.VMEM((RPW,), jnp.float32),
            pltpu.VMEM((RPW,), jnp.float32),
            pltpu.SemaphoreType.DMA,
        ],
    )(_body)
    bias_flat = item_biases.reshape(-1)
    return k(u, i, j, user_factors, item_factors, bias_flat)
